# Initial kernel scaffold; baseline (speedup 1.0000x reference)
import jax
import jax.numpy as jnp
from jax.experimental import pallas as pl


def kernel(x):
    raise NotImplementedError("write your pallas kernel here")


# speedup vs baseline: 1.7249x; 1.7249x over previous
import jax
import jax.numpy as jnp
from jax.experimental import pallas as pl
from jax.experimental.pallas import tpu as pltpu

_K = 1024
_D = 384
_N_ITERS = 10
_TN = 1152


def _assign_tile(pts, centers, c2_row):
    dot = jax.lax.dot_general(pts, centers, (((1,), (1,)), ((), ())),
                              preferred_element_type=jnp.float32)
    p2 = jnp.sum(pts * pts, axis=1, keepdims=True)
    d2 = (p2 - 2.0 * dot) + c2_row
    mind = jnp.min(d2, axis=1, keepdims=True)
    idx = jax.lax.broadcasted_iota(jnp.int32, d2.shape, 1)
    assign = jnp.min(jnp.where(d2 == mind, idx, _K), axis=1, keepdims=True)
    return assign


def _iter_kernel(pts_ref, centers_ref, out_ref, sums_ref, counts_ref, c2_ref):
    t = pl.program_id(0)
    nt = pl.num_programs(0)

    @pl.when(t == 0)
    def _init():
        sums_ref[...] = jnp.zeros_like(sums_ref)
        counts_ref[...] = jnp.zeros_like(counts_ref)
        c = centers_ref[...]
        c2_ref[...] = jnp.sum(c * c, axis=1)[None, :]

    pts = pts_ref[...]
    centers = centers_ref[...]
    assign = _assign_tile(pts, centers, c2_ref[...])
    assign_row = assign.reshape(1, _TN)
    kidx = jax.lax.broadcasted_iota(jnp.int32, (_K, _TN), 0)
    onehot_t = (kidx == assign_row).astype(jnp.float32)
    sums_ref[...] += jax.lax.dot_general(
        onehot_t, pts, (((1,), (0,)), ((), ())),
        precision=jax.lax.Precision.HIGHEST,
        preferred_element_type=jnp.float32)
    counts_ref[...] += jnp.sum(onehot_t, axis=1, keepdims=True)

    @pl.when(t == nt - 1)
    def _fin():
        cnt = jnp.maximum(counts_ref[...], 1.0)
        out_ref[...] = sums_ref[...] / cnt


def _final_kernel(pts_ref, centers_ref, out_ref, c2_ref):
    t = pl.program_id(0)

    @pl.when(t == 0)
    def _init():
        c = centers_ref[...]
        c2_ref[...] = jnp.sum(c * c, axis=1)[None, :]

    pts = pts_ref[...]
    centers = centers_ref[...]
    assign = _assign_tile(pts, centers, c2_ref[...])
    kidx = jax.lax.broadcasted_iota(jnp.int32, (_TN, _K), 1)
    out_ref[...] = (kidx == assign).astype(jnp.float32)


def _one_iter(pts, centers):
    nt = pts.shape[0] // _TN
    return pl.pallas_call(
        _iter_kernel,
        grid=(nt,),
        in_specs=[
            pl.BlockSpec((_TN, _D), lambda t: (t, 0)),
            pl.BlockSpec((_K, _D), lambda t: (0, 0)),
        ],
        out_specs=pl.BlockSpec((_K, _D), lambda t: (0, 0)),
        out_shape=jax.ShapeDtypeStruct((_K, _D), jnp.float32),
        scratch_shapes=[
            pltpu.VMEM((_K, _D), jnp.float32),
            pltpu.VMEM((_K, 1), jnp.float32),
            pltpu.VMEM((1, _K), jnp.float32),
        ],
    )(pts, centers)


def _final_assign(pts, centers):
    nt = pts.shape[0] // _TN
    return pl.pallas_call(
        _final_kernel,
        grid=(nt,),
        in_specs=[
            pl.BlockSpec((_TN, _D), lambda t: (t, 0)),
            pl.BlockSpec((_K, _D), lambda t: (0, 0)),
        ],
        out_specs=pl.BlockSpec((_TN, _K), lambda t: (t, 0)),
        out_shape=jax.ShapeDtypeStruct((pts.shape[0], _K), jnp.float32),
        scratch_shapes=[
            pltpu.VMEM((1, _K), jnp.float32),
        ],
    )(pts, centers)


def kernel(x):
    B, T, d = x.shape
    pts = x.reshape(B * T, d)
    centers = pts[:_K]
    centers = jax.lax.fori_loop(
        0, _N_ITERS, lambda i, c: _one_iter(pts, c), centers)
    onehot = _final_assign(pts, centers)
    return onehot.reshape(B, T, _K)


# speedup vs baseline: 1.8345x; 1.0635x over previous
import jax
import jax.numpy as jnp
from jax.experimental import pallas as pl
from jax.experimental.pallas import tpu as pltpu

_K = 1024
_D = 384
_N_ITERS = 10
_TN = 1152


def _kmeans_kernel(pts_ref, out_ref, centers_ref, sums_ref, counts_ref, c2_ref):
    i = pl.program_id(0)
    t = pl.program_id(1)
    nt = pl.num_programs(1)

    @pl.when(jnp.logical_and(i == 0, t == 0))
    def _init_centers():
        centers_ref[...] = pts_ref[:_K, :]

    @pl.when(t == 0)
    def _start_iter():
        sums_ref[...] = jnp.zeros_like(sums_ref)
        counts_ref[...] = jnp.zeros_like(counts_ref)
        c = centers_ref[...]
        c2_ref[...] = jnp.sum(c * c, axis=1)[None, :]

    pts = pts_ref[...]
    centers = centers_ref[...]
    dot = jax.lax.dot_general(pts, centers, (((1,), (1,)), ((), ())),
                              preferred_element_type=jnp.float32)
    p2 = jnp.sum(pts * pts, axis=1, keepdims=True)
    d2 = (p2 - 2.0 * dot) + c2_ref[...]
    mind = jnp.min(d2, axis=1, keepdims=True)
    idx = jax.lax.broadcasted_iota(jnp.int32, d2.shape, 1)
    assign = jnp.min(jnp.where(d2 == mind, idx, _K), axis=1, keepdims=True)

    @pl.when(i < _N_ITERS)
    def _accumulate():
        assign_row = assign.reshape(1, _TN)
        kidx = jax.lax.broadcasted_iota(jnp.int32, (_K, _TN), 0)
        onehot_t = (kidx == assign_row).astype(jnp.float32)
        sums_ref[...] += jax.lax.dot_general(
            onehot_t, pts, (((1,), (0,)), ((), ())),
            precision=jax.lax.Precision.HIGHEST,
            preferred_element_type=jnp.float32)
        counts_ref[...] += jnp.sum(onehot_t, axis=1, keepdims=True)

    @pl.when(jnp.logical_and(i < _N_ITERS, t == nt - 1))
    def _end_iter():
        cnt = jnp.maximum(counts_ref[...], 1.0)
        centers_ref[...] = sums_ref[...] / cnt

    @pl.when(i == _N_ITERS)
    def _emit():
        kidx2 = jax.lax.broadcasted_iota(jnp.int32, (_TN, _K), 1)
        out_ref[...] = (kidx2 == assign).astype(jnp.float32)


def kernel(x):
    B, T, d = x.shape
    n = B * T
    pts = x.reshape(n, d)
    nt = n // _TN
    onehot = pl.pallas_call(
        _kmeans_kernel,
        grid=(_N_ITERS + 1, nt),
        in_specs=[pl.BlockSpec((_TN, _D), lambda i, t: (t, 0))],
        out_specs=pl.BlockSpec(
            (_TN, _K), lambda i, t: (jnp.where(i < _N_ITERS, 0, t), 0)),
        out_shape=jax.ShapeDtypeStruct((n, _K), jnp.float32),
        scratch_shapes=[
            pltpu.VMEM((_K, _D), jnp.float32),
            pltpu.VMEM((_K, _D), jnp.float32),
            pltpu.VMEM((_K, 1), jnp.float32),
            pltpu.VMEM((1, _K), jnp.float32),
        ],
    )(pts)
    return onehot.reshape(B, T, _K)


# speedup vs baseline: 3.0507x; 1.6630x over previous
import jax
import jax.numpy as jnp
from jax.experimental import pallas as pl
from jax.experimental.pallas import tpu as pltpu

_K = 1024
_D = 384
_N_ITERS = 10
_TN = 2304


def _kmeans_kernel(pts_ref, out_ref, centers_ref, sums_ref, counts_ref, c2_ref):
    i = pl.program_id(0)
    t = pl.program_id(1)
    nt = pl.num_programs(1)

    @pl.when(jnp.logical_and(i == 0, t == 0))
    def _init_centers():
        centers_ref[...] = pts_ref[:_K, :]

    @pl.when(t == 0)
    def _start_iter():
        sums_ref[...] = jnp.zeros_like(sums_ref)
        counts_ref[...] = jnp.zeros_like(counts_ref)
        c = centers_ref[...]
        c2_ref[...] = jnp.sum(c * c, axis=1)[None, :]

    pts = pts_ref[...]
    centers = centers_ref[...]
    dot = jax.lax.dot_general(pts, centers, (((1,), (1,)), ((), ())),
                              preferred_element_type=jnp.float32)
    p2 = jnp.sum(pts * pts, axis=1, keepdims=True)
    d2 = (p2 - 2.0 * dot) + c2_ref[...]
    mind = jnp.min(d2, axis=1, keepdims=True)
    idx = jax.lax.broadcasted_iota(jnp.int32, d2.shape, 1)
    assign = jnp.min(jnp.where(d2 == mind, idx, _K), axis=1, keepdims=True)

    @pl.when(i < _N_ITERS)
    def _accumulate():
        assign_row = assign.reshape(1, _TN)
        kidx = jax.lax.broadcasted_iota(jnp.int32, (_K, _TN), 0)
        onehot16 = (kidx == assign_row).astype(jnp.bfloat16)
        hi = pts.astype(jnp.bfloat16)
        r1 = pts - hi.astype(jnp.float32)
        mid = r1.astype(jnp.bfloat16)
        lo = (r1 - mid.astype(jnp.float32)).astype(jnp.bfloat16)

        def dn(a, b):
            return jax.lax.dot_general(
                a, b, (((1,), (0,)), ((), ())),
                preferred_element_type=jnp.float32)

        sums_ref[...] += (dn(onehot16, hi) + dn(onehot16, mid)) + dn(onehot16, lo)
        counts_ref[...] += jnp.sum(
            onehot16.astype(jnp.float32), axis=1, keepdims=True)

    @pl.when(jnp.logical_and(i < _N_ITERS, t == nt - 1))
    def _end_iter():
        cnt = jnp.maximum(counts_ref[...], 1.0)
        centers_ref[...] = sums_ref[...] / cnt

    @pl.when(i == _N_ITERS)
    def _emit():
        kidx2 = jax.lax.broadcasted_iota(jnp.int32, (_TN, _K), 1)
        out_ref[...] = (kidx2 == assign).astype(jnp.float32)


def kernel(x):
    B, T, d = x.shape
    n = B * T
    pts = x.reshape(n, d)
    nt = n // _TN
    onehot = pl.pallas_call(
        _kmeans_kernel,
        grid=(_N_ITERS + 1, nt),
        in_specs=[pl.BlockSpec((_TN, _D), lambda i, t: (t, 0))],
        out_specs=pl.BlockSpec(
            (_TN, _K), lambda i, t: (jnp.where(i < _N_ITERS, 0, t), 0)),
        out_shape=jax.ShapeDtypeStruct((n, _K), jnp.float32),
        scratch_shapes=[
            pltpu.VMEM((_K, _D), jnp.float32),
            pltpu.VMEM((_K, _D), jnp.float32),
            pltpu.VMEM((_K, 1), jnp.float32),
            pltpu.VMEM((1, _K), jnp.float32),
        ],
    )(pts)
    return onehot.reshape(B, T, _K)


# speedup vs baseline: 3.3834x; 1.1091x over previous
import jax
import jax.numpy as jnp
from jax.experimental import pallas as pl
from jax.experimental.pallas import tpu as pltpu

_K = 1024
_D = 384
_N_ITERS = 10
_TN = 2304


def _kmeans_kernel(pts_ref, out_ref, centers_ref, sums_ref, counts_ref, c2_ref):
    i = pl.program_id(0)
    t = pl.program_id(1)
    nt = pl.num_programs(1)

    @pl.when(jnp.logical_and(i == 0, t == 0))
    def _init_centers():
        centers_ref[...] = pts_ref[:_K, :]

    @pl.when(t == 0)
    def _start_iter():
        sums_ref[...] = jnp.zeros_like(sums_ref)
        counts_ref[...] = jnp.zeros_like(counts_ref)
        c = centers_ref[...]
        c2_ref[...] = jnp.sum(c * c, axis=1)[None, :]

    pts = pts_ref[...]
    centers = centers_ref[...]
    dot = jax.lax.dot_general(pts, centers, (((1,), (1,)), ((), ())),
                              preferred_element_type=jnp.float32)
    p2 = jnp.sum(pts * pts, axis=1, keepdims=True)
    d2 = (p2 - 2.0 * dot) + c2_ref[...]
    mind = jnp.min(d2, axis=1, keepdims=True)
    idx = jax.lax.broadcasted_iota(jnp.int32, d2.shape, 1)
    assign = jnp.min(jnp.where(d2 == mind, idx, _K), axis=1, keepdims=True)

    @pl.when(i < _N_ITERS)
    def _accumulate():
        assign_row = assign.reshape(1, _TN)
        kidx = jax.lax.broadcasted_iota(jnp.int32, (_K, _TN), 0)
        onehot16 = (kidx == assign_row).astype(jnp.bfloat16)
        hi = pts.astype(jnp.bfloat16)
        r1 = pts - hi.astype(jnp.float32)
        mid = r1.astype(jnp.bfloat16)
        lo = (r1 - mid.astype(jnp.float32)).astype(jnp.bfloat16)

        rhs = jnp.concatenate([hi, mid, lo], axis=1)
        s = jax.lax.dot_general(
            onehot16, rhs, (((1,), (0,)), ((), ())),
            preferred_element_type=jnp.float32)
        sums_ref[...] += (s[:, :_D] + s[:, _D:2 * _D]) + s[:, 2 * _D:]
        counts_ref[...] += jnp.sum(
            onehot16.astype(jnp.float32), axis=1, keepdims=True)

    @pl.when(jnp.logical_and(i < _N_ITERS, t == nt - 1))
    def _end_iter():
        cnt = jnp.maximum(counts_ref[...], 1.0)
        centers_ref[...] = sums_ref[...] / cnt

    @pl.when(i == _N_ITERS)
    def _emit():
        kidx2 = jax.lax.broadcasted_iota(jnp.int32, (_TN, _K), 1)
        out_ref[...] = (kidx2 == assign).astype(jnp.float32)


def kernel(x):
    B, T, d = x.shape
    n = B * T
    pts = x.reshape(n, d)
    nt = n // _TN
    onehot = pl.pallas_call(
        _kmeans_kernel,
        grid=(_N_ITERS + 1, nt),
        in_specs=[pl.BlockSpec((_TN, _D), lambda i, t: (t, 0))],
        out_specs=pl.BlockSpec(
            (_TN, _K), lambda i, t: (jnp.where(i < _N_ITERS, 0, t), 0)),
        out_shape=jax.ShapeDtypeStruct((n, _K), jnp.float32),
        scratch_shapes=[
            pltpu.VMEM((_K, _D), jnp.float32),
            pltpu.VMEM((_K, _D), jnp.float32),
            pltpu.VMEM((_K, 1), jnp.float32),
            pltpu.VMEM((1, _K), jnp.float32),
        ],
    )(pts)
    return onehot.reshape(B, T, _K)
